# baseline (speedup 1.0000x reference)
import jax
import jax.numpy as jnp
from jax import lax
from jax.experimental import pallas as pl
from jax.experimental.pallas import tpu as pltpu
from jax.experimental.pallas import tpu_sc as plsc

TABLE_SIZE = 500000
HIDDEN = 768
NGRAM = 3
LANES = 16

_E = (1, 8192, 108864, 313888, 370496)
_POW64_MOD = 51616

NUM_CORES = 2
NUM_SUBCORES = 16
NUM_TILES = NUM_CORES * NUM_SUBCORES
TOK_PER_TILE = 256
CHUNK = 32
NCHUNK = TOK_PER_TILE // CHUNK
JCH = HIDDEN // LANES
WBLK = 8


def _hash_vec(win, mlv, cc):
  l = [jnp.zeros((LANES,), jnp.uint32) for _ in range(4)]
  for k in range(NGRAM):
    idv = win[pl.ds(8 + cc * LANES - k, LANES)].astype(jnp.uint32)
    carry = jnp.zeros((LANES,), jnp.uint32)
    for j in range(4):
      p = idv * mlv[k * 4 + j, :] + carry
      l[j] = l[j] ^ (p & jnp.uint32(0xFFFF))
      carry = p >> jnp.uint32(16)
  d0 = l[0] & jnp.uint32(0x1FFF)
  d1 = ((l[0] >> jnp.uint32(13)) | (l[1] << jnp.uint32(3))) & jnp.uint32(0x1FFF)
  d2 = ((l[1] >> jnp.uint32(10)) | (l[2] << jnp.uint32(6))) & jnp.uint32(0x1FFF)
  d3 = ((l[2] >> jnp.uint32(7)) | (l[3] << jnp.uint32(9))) & jnp.uint32(0x1FFF)
  d4 = l[3] >> jnp.uint32(4)
  r = (d4 * jnp.uint32(_E[4])) % jnp.uint32(TABLE_SIZE)
  for dj, ej in ((d3, _E[3]), (d2, _E[2]), (d1, _E[1]), (d0, _E[0])):
    r = (r + dj * jnp.uint32(ej)) % jnp.uint32(TABLE_SIZE)
  neg = (l[3] >> jnp.uint32(15)).astype(jnp.int32)
  ri = r.astype(jnp.int32) - neg * jnp.int32(_POW64_MOD)
  return jnp.where(ri < 0, ri + jnp.int32(TABLE_SIZE), ri)


def _engram_body(ids_h, emb_h, hid_h, ml_h, w_h, bv_h, out_h,
                 win, mlv, wv, bvv, hashes, rowb0, rowb1, hidb0, hidb1,
                 accb, gates,
                 gsem0, gsem1, hsem0, hsem1, osem0, osem1):
  cid = lax.axis_index("c")
  sid = lax.axis_index("s")
  wid = sid * NUM_CORES + cid
  base = wid * TOK_PER_TILE
  rowbs = (rowb0, rowb1)
  hidbs = (hidb0, hidb1)
  gsems = (gsem0, gsem1)
  hsems = (hsem0, hsem1)
  osems = (osem0, osem1)

  pltpu.sync_copy(ml_h, mlv)
  pltpu.sync_copy(w_h, wv)
  pltpu.sync_copy(bv_h, bvv)

  win[pl.ds(0, LANES)] = jnp.zeros((LANES,), jnp.int32)
  pltpu.sync_copy(ids_h.at[pl.ds(base, TOK_PER_TILE)],
                  win.at[pl.ds(8, TOK_PER_TILE)])

  @pl.when(wid % 8 != 0)
  def _():
    pltpu.sync_copy(ids_h.at[pl.ds(base - 8, 8)], win.at[pl.ds(0, 8)])

  def hash_body(cc, carry):
    hashes[pl.ds(cc * LANES, LANES)] = _hash_vec(win, mlv, cc)
    return carry

  with jax.named_scope("hash_phase"):
    lax.fori_loop(jnp.int32(0), jnp.int32(TOK_PER_TILE // LANES), hash_body,
                  jnp.int32(0))

  def start_in(ch, bi):
    pltpu.async_copy(emb_h.at[hashes.at[pl.ds(ch * CHUNK, CHUNK)]],
                     rowbs[bi], gsems[bi])
    pltpu.async_copy(hid_h.at[pl.ds(base + ch * CHUNK, CHUNK)],
                     hidbs[bi], hsems[bi])

  def wait_in(bi):
    with jax.named_scope("wait_in"):
      pltpu.make_async_copy(hid_h.at[pl.ds(0, CHUNK)], rowbs[bi],
                            gsems[bi]).wait()
      pltpu.make_async_copy(hid_h.at[pl.ds(0, CHUNK)], hidbs[bi],
                            hsems[bi]).wait()

  def start_out(ch, bi):
    pltpu.async_copy(rowbs[bi],
                     out_h.at[pl.ds(base + ch * CHUNK, CHUNK)], osems[bi])

  def wait_out(bi):
    with jax.named_scope("wait_out"):
      pltpu.make_async_copy(rowbs[bi], out_h.at[pl.ds(0, CHUNK)],
                            osems[bi]).wait()

  def compute(bi):
    rb = rowbs[bi]
    hb = hidbs[bi]
    for g in range(0, JCH, WBLK):
      wh = [wv[pl.ds((g + i) * LANES, LANES)] for i in range(WBLK)]
      wm = [wv[pl.ds(HIDDEN + (g + i) * LANES, LANES)] for i in range(WBLK)]

      def tok_a(u, carry, g=g, wh=wh, wm=wm):
        for v in range(8):
          t = u * 8 + jnp.int32(v)
          terms = []
          for i in range(WBLK):
            terms.append(hb[t, pl.ds((g + i) * LANES, LANES)] * wh[i])
            terms.append(rb[t, pl.ds((g + i) * LANES, LANES)] * wm[i])
          while len(terms) > 1:
            terms = [terms[k] + terms[k + 1]
                     for k in range(0, len(terms) - 1, 2)] + (
                         [terms[-1]] if len(terms) % 2 else [])
          if g == 0:
            accb[pl.ds(t * LANES, LANES)] = terms[0]
          else:
            accb[pl.ds(t * LANES, LANES)] = (
                accb[pl.ds(t * LANES, LANES)] + terms[0])
        return carry

      with jax.named_scope("pass_a"):
        lax.fori_loop(jnp.int32(0), jnp.int32(CHUNK // 8), tok_a,
                      jnp.int32(0))

    ivec = lax.iota(jnp.int32, LANES) * jnp.int32(LANES)
    for h in range(CHUNK // LANES):
      with jax.named_scope("pass_b1"):
        cols = [plsc.load_gather(
            accb, [ivec + jnp.int32(h * LANES * LANES + l)])
            for l in range(LANES)]
        while len(cols) > 1:
          cols = [cols[k] + cols[k + 1] for k in range(0, len(cols), 2)]
        acc16 = bvv[...] + cols[0]
        gates[pl.ds(h * LANES, LANES)] = 1.0 / (1.0 + jnp.exp(-acc16))

    def tok_b(u, carry, rb=rb):
      t0 = u * 2
      t1 = u * 2 + jnp.int32(1)
      g0 = plsc.load_gather(gates, [jnp.zeros((LANES,), jnp.int32) + t0])
      g1 = plsc.load_gather(gates, [jnp.zeros((LANES,), jnp.int32) + t1])
      for tt, gg in ((t0, g0), (t1, g1)):
        for j in range(JCH):
          rb[tt, pl.ds(j * LANES, LANES)] = (
              rb[tt, pl.ds(j * LANES, LANES)] * gg)
      return carry

    with jax.named_scope("pass_b2"):
      lax.fori_loop(jnp.int32(0), jnp.int32(CHUNK // 2), tok_b,
                    jnp.int32(0))

  start_in(0, 0)

  def chunk_pair(gp, carry):
    c0 = gp * 2
    c1 = c0 + 1

    @pl.when(c0 >= 2)
    def _():
      wait_out(0)
    start_in(c1, 1)
    wait_in(0)
    compute(0)
    start_out(c0, 0)

    @pl.when(c1 >= 2)
    def _():
      wait_out(1)

    @pl.when(c1 + 1 < NCHUNK)
    def _():
      start_in(c1 + 1, 0)
    wait_in(1)
    compute(1)
    start_out(c1, 1)
    return carry

  lax.fori_loop(jnp.int32(0), jnp.int32(NCHUNK // 2), chunk_pair,
                jnp.int32(0))
  wait_out(0)
  wait_out(1)


_SCRATCH_TYPES = [
    pltpu.VMEM((8 + TOK_PER_TILE + 8,), jnp.int32),
    pltpu.VMEM((NGRAM * 4, LANES), jnp.uint32),
    pltpu.VMEM((2 * HIDDEN,), jnp.float32),
    pltpu.VMEM((LANES,), jnp.float32),
    pltpu.VMEM((TOK_PER_TILE,), jnp.int32),
    pltpu.VMEM((CHUNK, HIDDEN), jnp.float32),
    pltpu.VMEM((CHUNK, HIDDEN), jnp.float32),
    pltpu.VMEM((CHUNK, HIDDEN), jnp.float32),
    pltpu.VMEM((CHUNK, HIDDEN), jnp.float32),
    pltpu.VMEM((CHUNK * LANES,), jnp.float32),
    pltpu.VMEM((CHUNK,), jnp.float32),
] + [pltpu.SemaphoreType.DMA for _ in range(6)]


@jax.jit
def _engram_sc(ids32, emb, hid, mlimb, w, bvec):
  ntok = ids32.shape[0]
  grid_kernel = pl.kernel(
      _engram_body,
      out_type=jax.ShapeDtypeStruct((ntok, HIDDEN), jnp.float32),
      mesh=plsc.VectorSubcoreMesh(
          core_axis_name="c", subcore_axis_name="s",
          num_cores=NUM_CORES, num_subcores=NUM_SUBCORES),
      scratch_types=_SCRATCH_TYPES,
      compiler_params=pltpu.CompilerParams(needs_layout_passes=False),
  )
  return grid_kernel(ids32, emb, hid, mlimb, w, bvec)


def kernel(hidden_states, input_ids, emb, gate_w, gate_b, multipliers):
  b, l, hdim = hidden_states.shape
  ids32 = input_ids.reshape(-1).astype(jnp.int32)
  hid = hidden_states.reshape(b * l, hdim)
  mu = multipliers.astype(jnp.uint64)
  shifts = jnp.arange(4, dtype=jnp.uint64) * jnp.uint64(16)
  limbs = ((mu[:, None] >> shifts[None, :]) & jnp.uint64(0xFFFF))
  mlimb = jnp.tile(limbs.astype(jnp.uint32).reshape(NGRAM * 4, 1),
                   (1, LANES))
  w = gate_w.reshape(2 * hdim)
  bvec = jnp.full((LANES,), gate_b.reshape(-1)[0].astype(jnp.float32),
                  jnp.float32)
  out = _engram_sc(ids32, emb, hid, mlimb, w, bvec)
  return out.reshape(b, l, hdim)


# speedup vs baseline: 1.0286x; 1.0286x over previous
import jax
import jax.numpy as jnp
from jax import lax
from jax.experimental import pallas as pl
from jax.experimental.pallas import tpu as pltpu
from jax.experimental.pallas import tpu_sc as plsc

TABLE_SIZE = 500000
HIDDEN = 768
NGRAM = 3
LANES = 16

_E = (1, 8192, 108864, 313888, 370496)
_POW64_MOD = 51616

NUM_CORES = 2
NUM_SUBCORES = 16
NUM_TILES = NUM_CORES * NUM_SUBCORES
TOK_PER_TILE = 256
CHUNK = 32
NCHUNK = TOK_PER_TILE // CHUNK
JCH = HIDDEN // LANES
WBLK = 8


def _hash_vec(win, mlv, cc):
  l = [jnp.zeros((LANES,), jnp.uint32) for _ in range(4)]
  for k in range(NGRAM):
    idv = win[pl.ds(8 + cc * LANES - k, LANES)].astype(jnp.uint32)
    carry = jnp.zeros((LANES,), jnp.uint32)
    for j in range(4):
      p = idv * mlv[k * 4 + j, :] + carry
      l[j] = l[j] ^ (p & jnp.uint32(0xFFFF))
      carry = p >> jnp.uint32(16)
  d0 = l[0] & jnp.uint32(0x1FFF)
  d1 = ((l[0] >> jnp.uint32(13)) | (l[1] << jnp.uint32(3))) & jnp.uint32(0x1FFF)
  d2 = ((l[1] >> jnp.uint32(10)) | (l[2] << jnp.uint32(6))) & jnp.uint32(0x1FFF)
  d3 = ((l[2] >> jnp.uint32(7)) | (l[3] << jnp.uint32(9))) & jnp.uint32(0x1FFF)
  d4 = l[3] >> jnp.uint32(4)
  r = (d4 * jnp.uint32(_E[4])) % jnp.uint32(TABLE_SIZE)
  for dj, ej in ((d3, _E[3]), (d2, _E[2]), (d1, _E[1]), (d0, _E[0])):
    r = (r + dj * jnp.uint32(ej)) % jnp.uint32(TABLE_SIZE)
  neg = (l[3] >> jnp.uint32(15)).astype(jnp.int32)
  ri = r.astype(jnp.int32) - neg * jnp.int32(_POW64_MOD)
  return jnp.where(ri < 0, ri + jnp.int32(TABLE_SIZE), ri)


def _engram_body(ids_h, emb_h, hid_h, ml_h, w_h, bv_h, out_h,
                 win, mlv, wv, bvv, hashes, rowb0, rowb1, hidb0, hidb1,
                 accb, gates,
                 gsem0, gsem1, hsem0, hsem1, osem0, osem1):
  cid = lax.axis_index("c")
  sid = lax.axis_index("s")
  wid = sid * NUM_CORES + cid
  base = wid * TOK_PER_TILE
  rowbs = (rowb0, rowb1)
  hidbs = (hidb0, hidb1)
  gsems = (gsem0, gsem1)
  hsems = (hsem0, hsem1)
  osems = (osem0, osem1)

  pltpu.sync_copy(ml_h, mlv)
  pltpu.sync_copy(w_h, wv)
  pltpu.sync_copy(bv_h, bvv)

  win[pl.ds(0, LANES)] = jnp.zeros((LANES,), jnp.int32)
  pltpu.sync_copy(ids_h.at[pl.ds(base, TOK_PER_TILE)],
                  win.at[pl.ds(8, TOK_PER_TILE)])

  @pl.when(wid % 8 != 0)
  def _():
    pltpu.sync_copy(ids_h.at[pl.ds(base - 8, 8)], win.at[pl.ds(0, 8)])

  def hash_body(cc, carry):
    hashes[pl.ds(cc * LANES, LANES)] = _hash_vec(win, mlv, cc)
    return carry

  with jax.named_scope("hash_phase"):
    lax.fori_loop(jnp.int32(0), jnp.int32(TOK_PER_TILE // LANES), hash_body,
                  jnp.int32(0))

  def start_in(ch, bi):
    pltpu.async_copy(emb_h.at[hashes.at[pl.ds(ch * CHUNK, CHUNK)]],
                     rowbs[bi], gsems[bi])
    pltpu.async_copy(hid_h.at[pl.ds(base + ch * CHUNK, CHUNK)],
                     hidbs[bi], hsems[bi])

  def wait_in(bi):
    with jax.named_scope("wait_in"):
      pltpu.make_async_copy(hid_h.at[pl.ds(0, CHUNK)], rowbs[bi],
                            gsems[bi]).wait()
      pltpu.make_async_copy(hid_h.at[pl.ds(0, CHUNK)], hidbs[bi],
                            hsems[bi]).wait()

  def start_out(ch, bi):
    pltpu.async_copy(rowbs[bi],
                     out_h.at[pl.ds(base + ch * CHUNK, CHUNK)], osems[bi])

  def wait_out(bi):
    with jax.named_scope("wait_out"):
      pltpu.make_async_copy(rowbs[bi], out_h.at[pl.ds(0, CHUNK)],
                            osems[bi]).wait()

  def compute(bi):
    rb = rowbs[bi]
    hb = hidbs[bi]
    for g in range(0, JCH, WBLK):
      wh = [wv[pl.ds((g + i) * LANES, LANES)] for i in range(WBLK)]
      wm = [wv[pl.ds(HIDDEN + (g + i) * LANES, LANES)] for i in range(WBLK)]

      def tok_a(u, carry, g=g, wh=wh, wm=wm):
        for v in range(4):
          t = u * 4 + jnp.int32(v)
          terms = []
          for i in range(WBLK):
            terms.append(hb[t, pl.ds((g + i) * LANES, LANES)] * wh[i])
            terms.append(rb[t, pl.ds((g + i) * LANES, LANES)] * wm[i])
          while len(terms) > 1:
            terms = [terms[k] + terms[k + 1]
                     for k in range(0, len(terms) - 1, 2)] + (
                         [terms[-1]] if len(terms) % 2 else [])
          if g == 0:
            accb[pl.ds(t * LANES, LANES)] = terms[0]
          else:
            accb[pl.ds(t * LANES, LANES)] = (
                accb[pl.ds(t * LANES, LANES)] + terms[0])
        return carry

      with jax.named_scope("pass_a"):
        lax.fori_loop(jnp.int32(0), jnp.int32(CHUNK // 4), tok_a,
                      jnp.int32(0))

    ivec = lax.iota(jnp.int32, LANES) * jnp.int32(LANES)
    for h in range(CHUNK // LANES):
      with jax.named_scope("pass_b1"):
        cols = [plsc.load_gather(
            accb, [ivec + jnp.int32(h * LANES * LANES + l)])
            for l in range(LANES)]
        while len(cols) > 1:
          cols = [cols[k] + cols[k + 1] for k in range(0, len(cols), 2)]
        acc16 = bvv[...] + cols[0]
        gates[pl.ds(h * LANES, LANES)] = 1.0 / (1.0 + jnp.exp(-acc16))

    def tok_b(u, carry, rb=rb):
      t0 = u * 2
      t1 = u * 2 + jnp.int32(1)
      g0 = plsc.load_gather(gates, [jnp.zeros((LANES,), jnp.int32) + t0])
      g1 = plsc.load_gather(gates, [jnp.zeros((LANES,), jnp.int32) + t1])
      for tt, gg in ((t0, g0), (t1, g1)):
        for j in range(JCH):
          rb[tt, pl.ds(j * LANES, LANES)] = (
              rb[tt, pl.ds(j * LANES, LANES)] * gg)
      return carry

    with jax.named_scope("pass_b2"):
      lax.fori_loop(jnp.int32(0), jnp.int32(CHUNK // 2), tok_b,
                    jnp.int32(0))

  start_in(0, 0)

  def chunk_pair(gp, carry):
    c0 = gp * 2
    c1 = c0 + 1

    @pl.when(c0 >= 2)
    def _():
      wait_out(0)
    start_in(c1, 1)
    wait_in(0)
    compute(0)
    start_out(c0, 0)

    @pl.when(c1 >= 2)
    def _():
      wait_out(1)

    @pl.when(c1 + 1 < NCHUNK)
    def _():
      start_in(c1 + 1, 0)
    wait_in(1)
    compute(1)
    start_out(c1, 1)
    return carry

  lax.fori_loop(jnp.int32(0), jnp.int32(NCHUNK // 2), chunk_pair,
                jnp.int32(0))
  wait_out(0)
  wait_out(1)


_SCRATCH_TYPES = [
    pltpu.VMEM((8 + TOK_PER_TILE + 8,), jnp.int32),
    pltpu.VMEM((NGRAM * 4, LANES), jnp.uint32),
    pltpu.VMEM((2 * HIDDEN,), jnp.float32),
    pltpu.VMEM((LANES,), jnp.float32),
    pltpu.VMEM((TOK_PER_TILE,), jnp.int32),
    pltpu.VMEM((CHUNK, HIDDEN), jnp.float32),
    pltpu.VMEM((CHUNK, HIDDEN), jnp.float32),
    pltpu.VMEM((CHUNK, HIDDEN), jnp.float32),
    pltpu.VMEM((CHUNK, HIDDEN), jnp.float32),
    pltpu.VMEM((CHUNK * LANES,), jnp.float32),
    pltpu.VMEM((CHUNK,), jnp.float32),
] + [pltpu.SemaphoreType.DMA for _ in range(6)]


@jax.jit
def _engram_sc(ids32, emb, hid, mlimb, w, bvec):
  ntok = ids32.shape[0]
  grid_kernel = pl.kernel(
      _engram_body,
      out_type=jax.ShapeDtypeStruct((ntok, HIDDEN), jnp.float32),
      mesh=plsc.VectorSubcoreMesh(
          core_axis_name="c", subcore_axis_name="s",
          num_cores=NUM_CORES, num_subcores=NUM_SUBCORES),
      scratch_types=_SCRATCH_TYPES,
      compiler_params=pltpu.CompilerParams(needs_layout_passes=False),
  )
  return grid_kernel(ids32, emb, hid, mlimb, w, bvec)


def kernel(hidden_states, input_ids, emb, gate_w, gate_b, multipliers):
  b, l, hdim = hidden_states.shape
  ids32 = input_ids.reshape(-1).astype(jnp.int32)
  hid = hidden_states.reshape(b * l, hdim)
  mu = multipliers.astype(jnp.uint64)
  shifts = jnp.arange(4, dtype=jnp.uint64) * jnp.uint64(16)
  limbs = ((mu[:, None] >> shifts[None, :]) & jnp.uint64(0xFFFF))
  mlimb = jnp.tile(limbs.astype(jnp.uint32).reshape(NGRAM * 4, 1),
                   (1, LANES))
  w = gate_w.reshape(2 * hdim)
  bvec = jnp.full((LANES,), gate_b.reshape(-1)[0].astype(jnp.float32),
                  jnp.float32)
  out = _engram_sc(ids32, emb, hid, mlimb, w, bvec)
  return out.reshape(b, l, hdim)
